# baseline (speedup 1.0000x reference)
import functools

import jax
import jax.numpy as jnp
from jax import lax
from jax.experimental import pallas as pl
from jax.experimental.pallas import tpu as pltpu
from jax.experimental.pallas import tpu_sc as plsc

_NW = 32
_NT = 16
_C = 80
_K8 = 4


def _round_up(x, m):
    return (x + m - 1) // m * m


@functools.lru_cache(maxsize=None)
def _make_seg_sum(NP, K, T, d):
    stripe = NP // _NT
    KC = K // _K8
    mesh = plsc.VectorSubcoreMesh(core_axis_name="c", subcore_axis_name="s")

    @functools.partial(
        pl.kernel,
        mesh=mesh,
        out_type=(
            jax.ShapeDtypeStruct((2, NP, d), jnp.float32),
            jax.ShapeDtypeStruct((2, NP), jnp.float32),
        ),
        scratch_types=[
            pltpu.VMEM_SHARED((NP, d), jnp.float32),
            pltpu.VMEM_SHARED((NP,), jnp.float32),
            pltpu.VMEM((2, _K8, _C), jnp.int32),
            pltpu.VMEM((2, _K8, _C), jnp.int32),
            pltpu.VMEM((4, _C, d), jnp.float32),
            pltpu.VMEM((_C,), jnp.float32),
            pltpu.SemaphoreType.DMA,
            pltpu.SemaphoreType.DMA,
            pltpu.SemaphoreType.DMA,
        ],
    )
    def seg_sum(table, gidx, sidx, z_d, z_1, sums_out, cnts_out,
                acc, cnt, gv, sv, gbuf, ones_v, sem_g, sem_s, sem_i):
        cid = lax.axis_index("c")
        sid = lax.axis_index("s")
        wid = cid * _NT + sid
        pltpu.sync_copy(z_d, acc.at[pl.ds(sid * stripe, stripe)])
        pltpu.sync_copy(z_1, cnt.at[pl.ds(sid * stripe, stripe)])
        for i in range(_C // 16):
            ones_v[pl.ds(i * 16, 16)] = jnp.ones((16,), jnp.float32)
        pltpu.sync_copy(gidx.at[wid, pl.ds(0, _K8)], gv.at[0])
        pltpu.sync_copy(sidx.at[wid, pl.ds(0, _K8)], sv.at[0])
        plsc.subcore_barrier()
        pltpu.async_copy(table.at[gv.at[0, 0]], gbuf.at[0], sem_g)
        pltpu.async_copy(table.at[gv.at[0, 1]], gbuf.at[1], sem_g)
        pltpu.async_copy(table.at[gv.at[0, 2]], gbuf.at[2], sem_g)

        def wait_scatter(a, r, slot):
            pltpu.make_async_copy(gbuf.at[slot], acc.at[sv.at[a, r]],
                                  sem_s).wait()
            pltpu.make_async_copy(ones_v, cnt.at[sv.at[a, r]], sem_s).wait()

        def chunk(cix, carry):
            a = cix % 2
            b = 1 - a
            for r in range(_K8):
                j0 = cix * _K8
                slot = (j0 + r) % 4
                pltpu.make_async_copy(table.at[gv.at[a, r]], gbuf.at[slot],
                                      sem_g).wait()
                if r == 0:
                    @pl.when(cix > 0)
                    def _():
                        wait_scatter(b, _K8 - 1, (j0 + 3) % 4)

                    @pl.when(cix < KC - 1)
                    def _():
                        pltpu.async_copy(
                            gidx.at[wid, pl.ds((cix + 1) * _K8, _K8)],
                            gv.at[b], sem_i)
                        pltpu.async_copy(
                            sidx.at[wid, pl.ds((cix + 1) * _K8, _K8)],
                            sv.at[b], sem_i)
                else:
                    wait_scatter(a, r - 1, (j0 + r + 3) % 4)
                if r < _K8 - 3:
                    pltpu.async_copy(table.at[gv.at[a, r + 3]],
                                     gbuf.at[(j0 + r + 3) % 4], sem_g)
                elif r == _K8 - 3:
                    @pl.when(cix < KC - 1)
                    def _():
                        pltpu.make_async_copy(
                            gidx.at[wid, pl.ds((cix + 1) * _K8, _K8)],
                            gv.at[b], sem_i).wait()
                        pltpu.make_async_copy(
                            sidx.at[wid, pl.ds((cix + 1) * _K8, _K8)],
                            sv.at[b], sem_i).wait()
                        pltpu.async_copy(table.at[gv.at[b, 0]],
                                         gbuf.at[(j0 + r + 3) % 4], sem_g)
                else:
                    @pl.when(cix < KC - 1)
                    def _():
                        pltpu.async_copy(table.at[gv.at[b, r - _K8 + 3]],
                                         gbuf.at[(j0 + r + 3) % 4], sem_g)
                pltpu.async_copy(gbuf.at[slot], acc.at[sv.at[a, r]], sem_s,
                                 add=True)
                pltpu.async_copy(ones_v, cnt.at[sv.at[a, r]], sem_s,
                                 add=True)
            return carry

        lax.fori_loop(0, KC, chunk, 0)
        wait_scatter((KC - 1) % 2, _K8 - 1, (K - 1) % 4)
        plsc.subcore_barrier()
        pltpu.sync_copy(acc.at[pl.ds(sid * stripe, stripe)],
                        sums_out.at[cid, pl.ds(sid * stripe, stripe)])
        pltpu.sync_copy(cnt.at[pl.ds(sid * stripe, stripe)],
                        cnts_out.at[cid, pl.ds(sid * stripe, stripe)])

    return seg_sum


def _mean_from_partials(sum_ref, cnt_ref):
    s = sum_ref[0] + sum_ref[1]
    c = cnt_ref[0] + cnt_ref[1]
    return s / jnp.maximum(c, 1.0)


def _edge_mlp_body(esum, ecnt, attr, w1, b1, w2, b2, out, *, d):
    mean = _mean_from_partials(esum, ecnt)
    h = (jnp.dot(mean, w1[0:d, :], preferred_element_type=jnp.float32)
         + jnp.dot(attr[...], w1[d:, :], preferred_element_type=jnp.float32)
         + b1[...])
    h = jnp.maximum(h, 0.0)
    out[...] = jnp.dot(h, w2[...], preferred_element_type=jnp.float32) + b2[...]


def _node_mlp_body(nsum, ncnt, xh, pw, pb, w1, b1, w2, b2, g, bb, out):
    msg = _mean_from_partials(nsum, ncnt)
    h = jnp.maximum(
        jnp.dot(msg, w1[...], preferred_element_type=jnp.float32) + b1[...], 0.0)
    nm = jnp.dot(h, w2[...], preferred_element_type=jnp.float32) + b2[...]
    z = (jnp.dot(xh[...], pw[...], preferred_element_type=jnp.float32)
         + pb[...] + nm)
    z = jnp.maximum(z, 0.0)
    mu = jnp.mean(z, axis=1, keepdims=True)
    zc = z - mu
    var = jnp.mean(zc * zc, axis=1, keepdims=True)
    out[...] = zc * lax.rsqrt(var + 1e-5) * g[...] + bb[...]


def _full(shape):
    return pl.BlockSpec(shape, lambda i: (0,) * len(shape))


def kernel(x_h, h2_edge_index, h2_edge_attr, proj_W, proj_b, e1_W, e1_b,
           e2_W, e2_b, n1_W, n1_b, n2_W, n2_b, ln_g, ln_b):
    H, d = x_h.shape
    E, de = h2_edge_attr.shape
    NI = h2_edge_index.shape[1]
    f32 = jnp.float32

    base = max(H, E)
    NP = _round_up(base + 8, 2048)
    K = _round_up(-(-NI // (_NW * _C)), _K8)
    npad = _NW * K * _C - NI
    spread = jnp.arange(npad, dtype=jnp.int32) % min(240, min(H, E), NP - base)

    fg = h2_edge_index[0].astype(jnp.int32)
    res = h2_edge_index[1].astype(jnp.int32)
    fg_g = jnp.concatenate([fg, spread]).reshape(_NW, K, _C)
    fg_s = jnp.concatenate([fg, base + spread]).reshape(_NW, K, _C)
    res_g = jnp.concatenate([res, spread]).reshape(_NW, K, _C)
    res_s = jnp.concatenate([res, base + spread]).reshape(_NW, K, _C)

    z_d = jnp.zeros((NP // _NT, d), f32)
    z_1 = jnp.zeros((NP // _NT,), f32)

    esum, ecnt = _make_seg_sum(NP, K, H, d)(x_h, fg_g, res_s, z_d, z_1)
    ecnt = ecnt.reshape(2, NP, 1)

    BR = 1000
    egrid = (E // BR,)
    edge_msg = pl.pallas_call(
        functools.partial(_edge_mlp_body, d=d),
        grid=egrid,
        in_specs=[
            pl.BlockSpec((2, BR, d), lambda i: (0, i, 0)),
            pl.BlockSpec((2, BR, 1), lambda i: (0, i, 0)),
            pl.BlockSpec((BR, de), lambda i: (i, 0)),
            _full((d + de, d)), _full((1, d)), _full((d, d)), _full((1, d)),
        ],
        out_specs=pl.BlockSpec((BR, d), lambda i: (i, 0)),
        out_shape=jax.ShapeDtypeStruct((E, d), f32),
    )(esum, ecnt, h2_edge_attr, e1_W, e1_b.reshape(1, d), e2_W,
      e2_b.reshape(1, d))

    nsum, ncnt = _make_seg_sum(NP, K, E, d)(edge_msg, res_g, fg_s, z_d, z_1)
    ncnt = ncnt.reshape(2, NP, 1)

    ngrid = (H // BR,)
    z = pl.pallas_call(
        _node_mlp_body,
        grid=ngrid,
        in_specs=[
            pl.BlockSpec((2, BR, d), lambda i: (0, i, 0)),
            pl.BlockSpec((2, BR, 1), lambda i: (0, i, 0)),
            pl.BlockSpec((BR, d), lambda i: (i, 0)),
            _full((d, d)), _full((1, d)),
            _full((d, d)), _full((1, d)),
            _full((d, d)), _full((1, d)),
            _full((1, d)), _full((1, d)),
        ],
        out_specs=pl.BlockSpec((BR, d), lambda i: (i, 0)),
        out_shape=jax.ShapeDtypeStruct((H, d), f32),
    )(nsum, ncnt, x_h, proj_W, proj_b.reshape(1, d),
      n1_W, n1_b.reshape(1, d), n2_W, n2_b.reshape(1, d),
      ln_g.reshape(1, d), ln_b.reshape(1, d))

    return z


# speedup vs baseline: 1.0275x; 1.0275x over previous
import functools

import jax
import jax.numpy as jnp
from jax import lax
from jax.experimental import pallas as pl
from jax.experimental.pallas import tpu as pltpu
from jax.experimental.pallas import tpu_sc as plsc

_NW = 32
_NT = 16
_C = 112
_K8 = 4


def _round_up(x, m):
    return (x + m - 1) // m * m


@functools.lru_cache(maxsize=None)
def _make_seg_sum(NP, K, T, d):
    stripe = NP // _NT
    KC = K // _K8
    mesh = plsc.VectorSubcoreMesh(core_axis_name="c", subcore_axis_name="s")

    @functools.partial(
        pl.kernel,
        mesh=mesh,
        out_type=(
            jax.ShapeDtypeStruct((2, NP, d), jnp.float32),
            jax.ShapeDtypeStruct((2, NP), jnp.float32),
        ),
        scratch_types=[
            pltpu.VMEM_SHARED((NP, d), jnp.float32),
            pltpu.VMEM_SHARED((NP,), jnp.float32),
            pltpu.VMEM((2, _K8, _C), jnp.int32),
            pltpu.VMEM((2, _K8, _C), jnp.int32),
            pltpu.VMEM((3, _C, d), jnp.float32),
            pltpu.VMEM((_C,), jnp.float32),
            pltpu.SemaphoreType.DMA,
            pltpu.SemaphoreType.DMA,
            pltpu.SemaphoreType.DMA,
        ],
    )
    def seg_sum(table, gidx, sidx, z_d, z_1, sums_out, cnts_out,
                acc, cnt, gv, sv, gbuf, ones_v, sem_g, sem_s, sem_i):
        cid = lax.axis_index("c")
        sid = lax.axis_index("s")
        wid = cid * _NT + sid
        pltpu.sync_copy(z_d, acc.at[pl.ds(sid * stripe, stripe)])
        pltpu.sync_copy(z_1, cnt.at[pl.ds(sid * stripe, stripe)])
        for i in range(_C // 16):
            ones_v[pl.ds(i * 16, 16)] = jnp.ones((16,), jnp.float32)
        pltpu.sync_copy(gidx.at[wid, pl.ds(0, _K8)], gv.at[0])
        pltpu.sync_copy(sidx.at[wid, pl.ds(0, _K8)], sv.at[0])
        plsc.subcore_barrier()
        pltpu.async_copy(table.at[gv.at[0, 0]], gbuf.at[0], sem_g)
        pltpu.async_copy(table.at[gv.at[0, 1]], gbuf.at[1], sem_g)

        def wait_scatter(a, r, slot):
            pltpu.make_async_copy(gbuf.at[slot], acc.at[sv.at[a, r]],
                                  sem_s).wait()
            pltpu.make_async_copy(ones_v, cnt.at[sv.at[a, r]], sem_s).wait()

        def chunk(cix, carry):
            a = cix % 2
            b = 1 - a
            for r in range(_K8):
                j0 = cix * _K8
                slot = (j0 + r) % 3
                if r == 0:
                    @pl.when(cix > 0)
                    def _():
                        wait_scatter(b, _K8 - 1, (j0 + 2) % 3)

                    @pl.when(cix < KC - 1)
                    def _():
                        pltpu.async_copy(
                            gidx.at[wid, pl.ds((cix + 1) * _K8, _K8)],
                            gv.at[b], sem_i)
                        pltpu.async_copy(
                            sidx.at[wid, pl.ds((cix + 1) * _K8, _K8)],
                            sv.at[b], sem_i)
                else:
                    wait_scatter(a, r - 1, (j0 + r + 2) % 3)
                if r < _K8 - 2:
                    pltpu.async_copy(table.at[gv.at[a, r + 2]],
                                     gbuf.at[(j0 + r + 2) % 3], sem_g)
                elif r == _K8 - 2:
                    @pl.when(cix < KC - 1)
                    def _():
                        pltpu.make_async_copy(
                            gidx.at[wid, pl.ds((cix + 1) * _K8, _K8)],
                            gv.at[b], sem_i).wait()
                        pltpu.make_async_copy(
                            sidx.at[wid, pl.ds((cix + 1) * _K8, _K8)],
                            sv.at[b], sem_i).wait()
                        pltpu.async_copy(table.at[gv.at[b, 0]],
                                         gbuf.at[(j0 + r + 2) % 3], sem_g)
                else:
                    @pl.when(cix < KC - 1)
                    def _():
                        pltpu.async_copy(table.at[gv.at[b, 1]],
                                         gbuf.at[(j0 + r + 2) % 3], sem_g)
                pltpu.make_async_copy(table.at[gv.at[a, r]], gbuf.at[slot],
                                      sem_g).wait()
                pltpu.async_copy(gbuf.at[slot], acc.at[sv.at[a, r]], sem_s,
                                 add=True)
                pltpu.async_copy(ones_v, cnt.at[sv.at[a, r]], sem_s,
                                 add=True)
            return carry

        lax.fori_loop(0, KC, chunk, 0)
        wait_scatter((KC - 1) % 2, _K8 - 1, (K - 1) % 3)
        plsc.subcore_barrier()
        pltpu.sync_copy(acc.at[pl.ds(sid * stripe, stripe)],
                        sums_out.at[cid, pl.ds(sid * stripe, stripe)])
        pltpu.sync_copy(cnt.at[pl.ds(sid * stripe, stripe)],
                        cnts_out.at[cid, pl.ds(sid * stripe, stripe)])

    return seg_sum


def _mean_from_partials(sum_ref, cnt_ref):
    s = sum_ref[0] + sum_ref[1]
    c = cnt_ref[0] + cnt_ref[1]
    return s / jnp.maximum(c, 1.0)


def _edge_mlp_body(esum, ecnt, attr, w1, b1, w2, b2, out, *, d):
    mean = _mean_from_partials(esum, ecnt)
    h = (jnp.dot(mean, w1[0:d, :], preferred_element_type=jnp.float32)
         + jnp.dot(attr[...], w1[d:, :], preferred_element_type=jnp.float32)
         + b1[...])
    h = jnp.maximum(h, 0.0)
    out[...] = jnp.dot(h, w2[...], preferred_element_type=jnp.float32) + b2[...]


def _node_mlp_body(nsum, ncnt, xh, pw, pb, w1, b1, w2, b2, g, bb, out):
    msg = _mean_from_partials(nsum, ncnt)
    h = jnp.maximum(
        jnp.dot(msg, w1[...], preferred_element_type=jnp.float32) + b1[...], 0.0)
    nm = jnp.dot(h, w2[...], preferred_element_type=jnp.float32) + b2[...]
    z = (jnp.dot(xh[...], pw[...], preferred_element_type=jnp.float32)
         + pb[...] + nm)
    z = jnp.maximum(z, 0.0)
    mu = jnp.mean(z, axis=1, keepdims=True)
    zc = z - mu
    var = jnp.mean(zc * zc, axis=1, keepdims=True)
    out[...] = zc * lax.rsqrt(var + 1e-5) * g[...] + bb[...]


def _full(shape):
    return pl.BlockSpec(shape, lambda i: (0,) * len(shape))


def kernel(x_h, h2_edge_index, h2_edge_attr, proj_W, proj_b, e1_W, e1_b,
           e2_W, e2_b, n1_W, n1_b, n2_W, n2_b, ln_g, ln_b):
    H, d = x_h.shape
    E, de = h2_edge_attr.shape
    NI = h2_edge_index.shape[1]
    f32 = jnp.float32

    base = max(H, E)
    NP = _round_up(base + 8, 2048)
    K = _round_up(-(-NI // (_NW * _C)), _K8)
    npad = _NW * K * _C - NI
    spread = jnp.arange(npad, dtype=jnp.int32) % min(240, min(H, E), NP - base)

    fg = h2_edge_index[0].astype(jnp.int32)
    res = h2_edge_index[1].astype(jnp.int32)
    fg_g = jnp.concatenate([fg, spread]).reshape(_NW, K, _C)
    fg_s = jnp.concatenate([fg, base + spread]).reshape(_NW, K, _C)
    res_g = jnp.concatenate([res, spread]).reshape(_NW, K, _C)
    res_s = jnp.concatenate([res, base + spread]).reshape(_NW, K, _C)

    z_d = jnp.zeros((NP // _NT, d), f32)
    z_1 = jnp.zeros((NP // _NT,), f32)

    esum, ecnt = _make_seg_sum(NP, K, H, d)(x_h, fg_g, res_s, z_d, z_1)
    ecnt = ecnt.reshape(2, NP, 1)

    BR = 1000
    egrid = (E // BR,)
    edge_msg = pl.pallas_call(
        functools.partial(_edge_mlp_body, d=d),
        grid=egrid,
        in_specs=[
            pl.BlockSpec((2, BR, d), lambda i: (0, i, 0)),
            pl.BlockSpec((2, BR, 1), lambda i: (0, i, 0)),
            pl.BlockSpec((BR, de), lambda i: (i, 0)),
            _full((d + de, d)), _full((1, d)), _full((d, d)), _full((1, d)),
        ],
        out_specs=pl.BlockSpec((BR, d), lambda i: (i, 0)),
        out_shape=jax.ShapeDtypeStruct((E, d), f32),
    )(esum, ecnt, h2_edge_attr, e1_W, e1_b.reshape(1, d), e2_W,
      e2_b.reshape(1, d))

    nsum, ncnt = _make_seg_sum(NP, K, E, d)(edge_msg, res_g, fg_s, z_d, z_1)
    ncnt = ncnt.reshape(2, NP, 1)

    ngrid = (H // BR,)
    z = pl.pallas_call(
        _node_mlp_body,
        grid=ngrid,
        in_specs=[
            pl.BlockSpec((2, BR, d), lambda i: (0, i, 0)),
            pl.BlockSpec((2, BR, 1), lambda i: (0, i, 0)),
            pl.BlockSpec((BR, d), lambda i: (i, 0)),
            _full((d, d)), _full((1, d)),
            _full((d, d)), _full((1, d)),
            _full((d, d)), _full((1, d)),
            _full((1, d)), _full((1, d)),
        ],
        out_specs=pl.BlockSpec((BR, d), lambda i: (i, 0)),
        out_shape=jax.ShapeDtypeStruct((H, d), f32),
    )(nsum, ncnt, x_h, proj_W, proj_b.reshape(1, d),
      n1_W, n1_b.reshape(1, d), n2_W, n2_b.reshape(1, d),
      ln_g.reshape(1, d), ln_b.reshape(1, d))

    return z
